# baseline (speedup 1.0000x reference)
import functools

import jax
import jax.numpy as jnp
from jax import lax
from jax.experimental import pallas as pl
from jax.experimental.pallas import tpu as pltpu

V = 100000
D = 128
L = 200

BLK = 3200
NS = 4
NB = -(-V // BLK)
NH = NB // NS
VT = V - (NB - 1) * BLK
assert NB % NS == 0


def _tc_body(ctx_ref, emb_ref, b_ref, *refs):
    w_refs = refs[0:NS]
    out_ref = refs[NS]
    rows_ref = refs[NS + 1]
    vm_ref = refs[NS + 2]
    acc_ref = refs[NS + 3]
    sem = refs[NS + 4]
    i = pl.program_id(0)

    @pl.when(i == 0)
    def _():
        copies = [
            pltpu.make_async_copy(
                emb_ref.at[pl.ds(ctx_ref[k], 1), :],
                rows_ref.at[pl.ds(k, 1), :],
                sem,
            )
            for k in range(L)
        ]
        for c in copies:
            c.start()
        for c in copies:
            c.wait()
        vsum = jnp.sum(rows_ref[...], axis=0, keepdims=True) * (1.0 / L)
        vm_ref[...] = lax.transpose(jnp.broadcast_to(vsum, (8, D)), (1, 0))

    vm = vm_ref[...]
    dn = (((1,), (0,)), ((), ()))
    for k in range(NS):
        lo_col = lax.dot_general(
            w_refs[k][...], vm, dn, preferred_element_type=jnp.float32
        )
        base = (i + k * NH) * BLK
        lo = lax.transpose(lo_col, (1, 0))[0:1, :]
        lo = lo + b_ref[pl.ds(base, BLK)].reshape(1, BLK)
        if k == NS - 1:
            @pl.when(i < NH - 1)
            def _(lo=lo, base=base):
                out_ref[:, pl.ds(base, BLK)] = lo

            @pl.when(i == NH - 1)
            def _(lo=lo, base=base):
                out_ref[:, pl.ds(base, VT)] = lo[:, :VT]
        else:
            out_ref[:, pl.ds(base, BLK)] = lo

    @pl.when(i == NH - 2)
    def _():
        parts = [
            out_ref[:, pl.ds(k * NH * BLK, (NH - 2) * BLK)] for k in range(NS)
        ]
        m1 = jnp.float32(-jnp.inf)
        for p in parts:
            m1 = jnp.maximum(m1, jnp.max(p))
        s1 = jnp.float32(0.0)
        for p in parts:
            s1 = s1 + jnp.sum(jnp.exp(p - m1))
        acc_ref[0] = m1
        acc_ref[1] = s1

    @pl.when(i == NH - 1)
    def _():
        fresh = []
        for k in range(NS):
            f0 = (k * NH + NH - 2) * BLK
            f1 = V if k == NS - 1 else (k + 1) * NH * BLK
            fresh.append(out_ref[:, pl.ds(f0, f1 - f0)])
        m2 = jnp.float32(-jnp.inf)
        for p in fresh:
            m2 = jnp.maximum(m2, jnp.max(p))
        s2 = jnp.float32(0.0)
        for p in fresh:
            s2 = s2 + jnp.sum(jnp.exp(p - m2))
        m1 = acc_ref[0]
        s1 = acc_ref[1]
        m = jnp.maximum(m1, m2)
        lse = m + jnp.log(s1 * jnp.exp(m1 - m) + s2 * jnp.exp(m2 - m))
        out_ref[...] = out_ref[...] - lse


def _mk_w_spec(k):
    return pl.BlockSpec((BLK, D), lambda i, k=k: (i + k * NH, 0))


_tc_cbow = pl.pallas_call(
    _tc_body,
    grid=(NH,),
    in_specs=(
        [
            pl.BlockSpec(memory_space=pltpu.SMEM),
            pl.BlockSpec(memory_space=pl.ANY),
            pl.BlockSpec((V,), lambda i: (0,)),
        ]
        + [_mk_w_spec(k) for k in range(NS)]
    ),
    out_specs=pl.BlockSpec((1, V), lambda i: (0, 0)),
    out_shape=jax.ShapeDtypeStruct((1, V), jnp.float32),
    scratch_shapes=[
        pltpu.VMEM((L, D), jnp.float32),
        pltpu.VMEM((D, 8), jnp.float32),
        pltpu.SMEM((2,), jnp.float32),
        pltpu.SemaphoreType.DMA,
    ],
    compiler_params=pltpu.CompilerParams(
        dimension_semantics=("arbitrary",)
    ),
)


def kernel(context, emb_table, W, b):
    context = context.astype(jnp.int32)
    return _tc_cbow(context, emb_table, b, *([W] * NS))


# speedup vs baseline: 1.0115x; 1.0115x over previous
import functools

import jax
import jax.numpy as jnp
from jax import lax
from jax.experimental import pallas as pl
from jax.experimental.pallas import tpu as pltpu

V = 100000
D = 128
L = 200

BLK = 3200
NS = 4
NB = -(-V // BLK)
NH = NB // NS
VT = V - (NB - 1) * BLK
assert NB % NS == 0


def _tc_body(ctx_ref, emb_ref, b_ref, *refs):
    w_refs = refs[0:NS]
    out_ref = refs[NS]
    rows_ref = refs[NS + 1]
    vm_ref = refs[NS + 2]
    acc_ref = refs[NS + 3]
    sem = refs[NS + 4]
    i = pl.program_id(0)

    @pl.when(i == 0)
    def _():
        copies = [
            pltpu.make_async_copy(
                emb_ref.at[pl.ds(ctx_ref[k], 1), :],
                rows_ref.at[pl.ds(k, 1), :],
                sem,
            )
            for k in range(L)
        ]
        for c in copies:
            c.start()
        for c in copies:
            c.wait()
        vsum = jnp.sum(rows_ref[...], axis=0, keepdims=True) * (1.0 / L)
        vm_ref[...] = lax.transpose(jnp.broadcast_to(vsum, (8, D)), (1, 0))

    vm = vm_ref[...]
    dn = (((1,), (0,)), ((), ()))
    for k in range(NS):
        lo_col = lax.dot_general(
            w_refs[k][...], vm, dn, preferred_element_type=jnp.float32
        )
        base = (i + k * NH) * BLK
        lo = lax.transpose(lo_col, (1, 0))[0:1, :]
        if k == NS - 1:
            @pl.when(i < NH - 1)
            def _(lo=lo, base=base):
                out_ref[:, pl.ds(base, BLK)] = (
                    lo + b_ref[pl.ds(base, BLK)].reshape(1, BLK)
                )

            @pl.when(i == NH - 1)
            def _(lo=lo, base=base):
                out_ref[:, pl.ds(base, VT)] = (
                    lo[:, :VT] + b_ref[pl.ds(base, VT)].reshape(1, VT)
                )
        else:
            out_ref[:, pl.ds(base, BLK)] = (
                lo + b_ref[pl.ds(base, BLK)].reshape(1, BLK)
            )

    @pl.when(i == NH - 2)
    def _():
        parts = [
            out_ref[:, pl.ds(k * NH * BLK, (NH - 2) * BLK)] for k in range(NS)
        ]
        m1 = jnp.float32(-jnp.inf)
        for p in parts:
            m1 = jnp.maximum(m1, jnp.max(p))
        s1 = jnp.float32(0.0)
        for p in parts:
            s1 = s1 + jnp.sum(jnp.exp(p - m1))
        acc_ref[0] = m1
        acc_ref[1] = s1

    @pl.when(i == NH - 1)
    def _():
        fresh = []
        for k in range(NS):
            f0 = (k * NH + NH - 2) * BLK
            f1 = V if k == NS - 1 else (k + 1) * NH * BLK
            fresh.append(out_ref[:, pl.ds(f0, f1 - f0)])
        m2 = jnp.float32(-jnp.inf)
        for p in fresh:
            m2 = jnp.maximum(m2, jnp.max(p))
        s2 = jnp.float32(0.0)
        for p in fresh:
            s2 = s2 + jnp.sum(jnp.exp(p - m2))
        m1 = acc_ref[0]
        s1 = acc_ref[1]
        m = jnp.maximum(m1, m2)
        lse = m + jnp.log(s1 * jnp.exp(m1 - m) + s2 * jnp.exp(m2 - m))
        out_ref[...] = out_ref[...] - lse


def _mk_w_spec(k):
    return pl.BlockSpec((BLK, D), lambda i, k=k: (i + k * NH, 0))


_tc_cbow = pl.pallas_call(
    _tc_body,
    grid=(NH,),
    in_specs=(
        [
            pl.BlockSpec(memory_space=pltpu.SMEM),
            pl.BlockSpec(memory_space=pl.ANY),
            pl.BlockSpec((V,), lambda i: (0,)),
        ]
        + [_mk_w_spec(k) for k in range(NS)]
    ),
    out_specs=pl.BlockSpec((1, V), lambda i: (0, 0)),
    out_shape=jax.ShapeDtypeStruct((1, V), jnp.float32),
    scratch_shapes=[
        pltpu.VMEM((L, D), jnp.float32),
        pltpu.VMEM((D, 8), jnp.float32),
        pltpu.SMEM((2,), jnp.float32),
        pltpu.SemaphoreType.DMA,
    ],
    compiler_params=pltpu.CompilerParams(
        dimension_semantics=("arbitrary",)
    ),
)


def kernel(context, emb_table, W, b):
    context = context.astype(jnp.int32)
    return _tc_cbow(context, emb_table, b, *([W] * NS))
